# Initial kernel scaffold; baseline (speedup 1.0000x reference)
import jax
import jax.numpy as jnp
from jax.experimental import pallas as pl


def kernel(x, edge_index, batch, W1, b1, g1, be1, rm1, rv1, W2, b2, g2, be2, rm2, rv2, W3, b3, g3, be3, rm3, rv3, mW1, mb1, mW2, mb2):
    raise NotImplementedError("write your pallas kernel here")


# speedup vs baseline: 10.2264x; 10.2264x over previous
import functools

import jax
import jax.numpy as jnp
from jax import lax
from jax.experimental import pallas as pl
from jax.experimental.pallas import tpu as pltpu
from jax.experimental.pallas import tpu_sc as plsc

N = 10000
E = 320000
F_IN = 128
H = 256
G = 64
EPS = 1e-5

NACC = 10112
STRIPE = NACC // 16
EPAD = 323584
CH_FS = EPAD // (16 * 128)
CH_ES = EPAD // (32 * 128)

_MESH = dict(core_axis_name="c", subcore_axis_name="s")


@functools.partial(
    pl.kernel,
    out_type=jax.ShapeDtypeStruct((2, NACC, 16), jnp.float32),
    mesh=plsc.VectorSubcoreMesh(**_MESH),
    scratch_types=[
        pltpu.VMEM((CH_ES, 128), jnp.int32),
        pltpu.VMEM((128, 16), jnp.float32),
        pltpu.VMEM_SHARED((NACC, 16), jnp.float32),
    ],
)
def _deg_sc(cols_hbm, zeros16_hbm, ones16_hbm, out_hbm, colv, onesv, dacc):
    c = lax.axis_index("c")
    s = lax.axis_index("s")
    wid = s * 2 + c

    pltpu.sync_copy(zeros16_hbm, dacc.at[pl.ds(s * STRIPE, STRIPE)])
    pltpu.sync_copy(cols_hbm.at[wid], colv)
    pltpu.sync_copy(ones16_hbm, onesv)
    plsc.subcore_barrier()

    def body(j, carry):
        pltpu.sync_copy(onesv, dacc.at[colv.at[j]], add=True)
        return carry

    lax.fori_loop(0, CH_ES, body, 0)
    plsc.subcore_barrier()
    pltpu.sync_copy(dacc.at[pl.ds(s * STRIPE, STRIPE)],
                    out_hbm.at[c, pl.ds(s * STRIPE, STRIPE)])


def _make_spmm(feature_split):
    chunks = CH_FS if feature_split else CH_ES

    @functools.partial(
        pl.kernel,
        out_type=jax.ShapeDtypeStruct((2, NACC, 128), jnp.float32),
        mesh=plsc.VectorSubcoreMesh(**_MESH),
        scratch_types=[
            pltpu.VMEM((CH_ES, 128), jnp.int32),
            pltpu.VMEM((CH_ES, 128), jnp.int32),
            pltpu.VMEM((128, 128), jnp.float32),
            pltpu.VMEM_SHARED((NACC, 128), jnp.float32),
            pltpu.SemaphoreType.DMA,
        ],
    )
    def spmm(hs_hbm, rows_hbm, cols_hbm, zeros_hbm, out_hbm, rowv, colv, buf,
             acc, sem):
        c = lax.axis_index("c")
        s = lax.axis_index("s")

        pltpu.sync_copy(zeros_hbm, acc.at[pl.ds(s * STRIPE, STRIPE)])
        plsc.subcore_barrier()

        def group(g, carry):
            if feature_split:
                pltpu.sync_copy(rows_hbm.at[c, s, g], rowv)
                pltpu.sync_copy(cols_hbm.at[s, g], colv)
            else:
                wid = s * 2 + c
                pltpu.sync_copy(rows_hbm.at[wid], rowv)
                pltpu.sync_copy(cols_hbm.at[wid], colv)

            def body(j, carry2):
                pltpu.async_copy(hs_hbm.at[rowv.at[j]], buf, sem).wait()
                pltpu.sync_copy(buf, acc.at[colv.at[j]], add=True)
                return carry2

            lax.fori_loop(0, CH_ES, body, 0)
            return carry

        lax.fori_loop(0, chunks // CH_ES, group, 0)
        plsc.subcore_barrier()
        pltpu.sync_copy(acc.at[pl.ds(s * STRIPE, STRIPE)],
                        out_hbm.at[c, pl.ds(s * STRIPE, STRIPE)])

    return spmm


_spmm_es = _make_spmm(False)
_spmm_fs = _make_spmm(True)


def _dinv_body(deg_ref, out_ref):
    out_ref[...] = lax.rsqrt(deg_ref[0] + deg_ref[1] + 1.0)


def _dinv_tc(degparts):
    return pl.pallas_call(
        _dinv_body,
        out_shape=jax.ShapeDtypeStruct((79, 128), jnp.float32),
    )(degparts)


def _hsx_body(x_ref, dinv_ref, o_ref):
    o_ref[...] = x_ref[...] * dinv_ref[...]


def _hsx_tc(x, dinv_n):
    nb = 1000
    return pl.pallas_call(
        _hsx_body,
        grid=(N // nb,),
        in_specs=[
            pl.BlockSpec((nb, F_IN), lambda i: (i, 0)),
            pl.BlockSpec((nb, 1), lambda i: (i, 0)),
        ],
        out_specs=pl.BlockSpec((nb, F_IN), lambda i: (i, 0)),
        out_shape=jax.ShapeDtypeStruct((N, F_IN), jnp.float32),
    )(x, dinv_n)


def _make_layer(first, out_hs):
    def body(acc_ref, hsp_ref, dinv_ref, w_ref, b_ref, g_ref, be_ref,
             rm_ref, rv_ref, o_ref):
        dinv = dinv_ref[...]
        if first:
            comb = acc_ref[0] + acc_ref[1] + hsp_ref[...]
        else:
            comb = (jnp.concatenate([acc_ref[0], acc_ref[1]], axis=1)
                    + jnp.concatenate([hsp_ref[0], hsp_ref[1]], axis=1))
        conv = dinv * comb
        y = jnp.dot(conv, w_ref[...], preferred_element_type=jnp.float32)
        t = ((y + b_ref[...] - rm_ref[...])
             * lax.rsqrt(rv_ref[...] + EPS) * g_ref[...] + be_ref[...])
        h = jnp.maximum(t, 0.0)
        if out_hs:
            hs = h * dinv
            o_ref[0] = hs[:, :128]
            o_ref[1] = hs[:, 128:]
        else:
            o_ref[...] = h

    nb = 1000
    fin = F_IN if first else H
    if first:
        hsp_spec = pl.BlockSpec((nb, F_IN), lambda i: (i, 0))
    else:
        hsp_spec = pl.BlockSpec((2, nb, 128), lambda i: (0, i, 0))
    if out_hs:
        out_spec = pl.BlockSpec((2, nb, 128), lambda i: (0, i, 0))
        out_shape = jax.ShapeDtypeStruct((2, N, 128), jnp.float32)
    else:
        out_spec = pl.BlockSpec((nb, H), lambda i: (i, 0))
        out_shape = jax.ShapeDtypeStruct((N, H), jnp.float32)

    def run(acc, hsp, dinv_n, w, b, g, be, rm, rv):
        return pl.pallas_call(
            body,
            grid=(N // nb,),
            in_specs=[
                pl.BlockSpec((2, nb, 128), lambda i: (0, i, 0)),
                hsp_spec,
                pl.BlockSpec((nb, 1), lambda i: (i, 0)),
                pl.BlockSpec((fin, H), lambda i: (0, 0)),
                pl.BlockSpec((1, H), lambda i: (0, 0)),
                pl.BlockSpec((1, H), lambda i: (0, 0)),
                pl.BlockSpec((1, H), lambda i: (0, 0)),
                pl.BlockSpec((1, H), lambda i: (0, 0)),
                pl.BlockSpec((1, H), lambda i: (0, 0)),
            ],
            out_specs=out_spec,
            out_shape=out_shape,
        )(acc, hsp, dinv_n, w, b, g, be, rm, rv)

    return run


_layer1 = _make_layer(True, True)
_layer2 = _make_layer(False, True)
_layer3 = _make_layer(False, False)


def _pool_body(h_ref, bf_ref, w1_ref, b1_ref, w2_ref, b2_ref, out_ref,
               pacc, cacc):
    i = pl.program_id(0)

    @pl.when(i == 0)
    def _():
        pacc[...] = jnp.zeros_like(pacc)
        cacc[...] = jnp.zeros_like(cacc)

    gids = lax.broadcasted_iota(jnp.int32, (1, G), 1).astype(jnp.float32)
    onehot = jnp.where(bf_ref[...] == gids, 1.0, 0.0)
    pacc[...] += lax.dot_general(onehot, h_ref[...], (((0,), (0,)), ((), ())),
                                 preferred_element_type=jnp.float32)
    ones = jnp.ones((onehot.shape[0], 128), jnp.float32)
    cacc[...] += lax.dot_general(onehot, ones, (((0,), (0,)), ((), ())),
                                 preferred_element_type=jnp.float32)

    @pl.when(i == pl.num_programs(0) - 1)
    def _():
        cnt = jnp.maximum(cacc[:, :1], 1.0)
        pooled = pacc[...] / cnt
        z = jnp.maximum(jnp.dot(pooled, w1_ref[...],
                                preferred_element_type=jnp.float32)
                        + b1_ref[...], 0.0)
        logit = jnp.sum(z * w2_ref[...], axis=1, keepdims=True) + b2_ref[...]
        out_ref[...] = logit


def _pool_tc(h3, batchf, mW1, mb1, mW2r, mb2):
    nb = 1000
    return pl.pallas_call(
        _pool_body,
        grid=(N // nb,),
        in_specs=[
            pl.BlockSpec((nb, H), lambda i: (i, 0)),
            pl.BlockSpec((nb, 1), lambda i: (i, 0)),
            pl.BlockSpec((H, H), lambda i: (0, 0)),
            pl.BlockSpec((1, H), lambda i: (0, 0)),
            pl.BlockSpec((1, H), lambda i: (0, 0)),
            pl.BlockSpec((1, 1), lambda i: (0, 0)),
        ],
        out_specs=pl.BlockSpec((G, 1), lambda i: (0, 0)),
        out_shape=jax.ShapeDtypeStruct((G, 1), jnp.float32),
        scratch_shapes=[
            pltpu.VMEM((G, H), jnp.float32),
            pltpu.VMEM((G, 128), jnp.float32),
        ],
    )(h3, batchf, mW1, mb1, mW2r, mb2)


def kernel(x, edge_index, batch, W1, b1, g1, be1, rm1, rv1, W2, b2, g2, be2,
           rm2, rv2, W3, b3, g3, be3, rm3, rv3, mW1, mb1, mW2, mb2):
    ei = edge_index.astype(jnp.int32)
    pad = EPAD - E
    rowp = jnp.concatenate([ei[0], jnp.zeros((pad,), jnp.int32)])
    colp = jnp.concatenate([ei[1], jnp.full((pad,), N, jnp.int32)])

    cols_es = colp.reshape(32, CH_ES, 128)
    rows_es = rowp.reshape(32, CH_ES, 128)
    rows_fs = jnp.stack([rowp, rowp + N]).reshape(2, 16, 2, CH_ES, 128)
    cols_fs = colp.reshape(16, 2, CH_ES, 128)

    zeros = jnp.zeros((STRIPE, 128), jnp.float32)
    zeros16 = jnp.zeros((STRIPE, 16), jnp.float32)
    ones16 = jnp.ones((128, 16), jnp.float32)

    degparts = _deg_sc(cols_es, zeros16, ones16)
    deg2d = degparts[:, :, 0].reshape(2, 79, 128)
    dinv = _dinv_tc(deg2d).reshape(NACC)[:N, None]

    hsx = _hsx_tc(x, dinv)

    b1r, g1r, be1r, rm1r, rv1r = (v.reshape(1, H) for v in (b1, g1, be1, rm1, rv1))
    b2r, g2r, be2r, rm2r, rv2r = (v.reshape(1, H) for v in (b2, g2, be2, rm2, rv2))
    b3r, g3r, be3r, rm3r, rv3r = (v.reshape(1, H) for v in (b3, g3, be3, rm3, rv3))

    acc1 = _spmm_es(hsx, rows_es, cols_es, zeros)
    hs1 = _layer1(acc1[:, :N], hsx, dinv, W1, b1r, g1r, be1r, rm1r, rv1r)

    acc2 = _spmm_fs(hs1.reshape(2 * N, 128), rows_fs, cols_fs, zeros)
    hs2 = _layer2(acc2[:, :N], hs1, dinv, W2, b2r, g2r, be2r, rm2r, rv2r)

    acc3 = _spmm_fs(hs2.reshape(2 * N, 128), rows_fs, cols_fs, zeros)
    h3 = _layer3(acc3[:, :N], hs2, dinv, W3, b3r, g3r, be3r, rm3r, rv3r)

    batchf = batch.astype(jnp.float32).reshape(N, 1)
    logit = _pool_tc(h3, batchf, mW1, mb1.reshape(1, H),
                     mW2.reshape(1, H), mb2.reshape(1, 1))
    return logit[:, 0]
